# Initial kernel scaffold; baseline (speedup 1.0000x reference)
import jax
import jax.numpy as jnp
from jax.experimental import pallas as pl


def kernel(ind, scales):
    raise NotImplementedError("write your pallas kernel here")


# speedup vs baseline: 1.2919x; 1.2919x over previous
import functools

import jax
import jax.numpy as jnp
from jax import lax
from jax.experimental import pallas as pl
from jax.experimental.pallas import tpu as pltpu
from jax.experimental.pallas import tpu_sc as plsc

_BATCH = 16384
_FIELDS = 26
_B = _BATCH * _FIELDS
_NC = 2
_NS = 16
_NW = _NC * _NS
_BPW = _B // _NW

_mesh = plsc.VectorSubcoreMesh(core_axis_name="c", subcore_axis_name="s")


@functools.partial(
    pl.kernel,
    mesh=_mesh,
    out_type=jax.ShapeDtypeStruct((_B,), jnp.float32),
    scratch_types=[
        pltpu.VMEM((_BPW,), jnp.int32),
        pltpu.VMEM((_BPW,), jnp.float32),
        pltpu.SemaphoreType.DMA,
    ],
)
def _gather_sc(idx_hbm, table_hbm, out_hbm, idx_v, vals_v, sem):
    wid = lax.axis_index("s") * _NC + lax.axis_index("c")
    base = wid * _BPW
    pltpu.sync_copy(idx_hbm.at[pl.ds(base, _BPW)], idx_v)
    pltpu.async_copy(table_hbm.at[idx_v], vals_v, sem).wait()
    pltpu.sync_copy(vals_v, out_hbm.at[pl.ds(base, _BPW)])


def kernel(ind, scales):
    flat = ind.reshape(_B).astype(jnp.int32)
    out = _gather_sc(flat, scales)
    return out.reshape(_BATCH, _FIELDS)


# speedup vs baseline: 1.3465x; 1.0423x over previous
import functools

import jax
import jax.numpy as jnp
from jax import lax
from jax.experimental import pallas as pl
from jax.experimental.pallas import tpu as pltpu
from jax.experimental.pallas import tpu_sc as plsc

_BATCH = 16384
_FIELDS = 26
_B = _BATCH * _FIELDS
_V = 1000000
_NC = 2
_NS = 16
_NW = _NC * _NS
_BPW = _B // _NW

_CHUNK = 62504
_TAIL = _V - 15 * _CHUNK
_SCHUNK = 13312
_NFULL = _CHUNK // _SCHUNK
_BTAIL = _CHUNK - _NFULL * _SCHUNK
_TTAIL = _TAIL - _NFULL * _SCHUNK

_mesh = plsc.VectorSubcoreMesh(core_axis_name="c", subcore_axis_name="s")


@functools.partial(
    pl.kernel,
    mesh=_mesh,
    out_type=jax.ShapeDtypeStruct((_B,), jnp.float32),
    scratch_types=[
        pltpu.VMEM((_BPW,), jnp.int32),
        pltpu.VMEM((_BPW,), jnp.float32),
        pltpu.VMEM((_SCHUNK,), jnp.float32),
        pltpu.VMEM_SHARED((_V,), jnp.float32),
        pltpu.SemaphoreType.DMA,
    ],
)
def _gather_sc(idx_hbm, table_hbm, out_hbm, idx_v, vals_v, stage_v, table_sp,
               sem):
    s = lax.axis_index("s")
    wid = s * _NC + lax.axis_index("c")
    base = wid * _BPW
    pltpu.sync_copy(idx_hbm.at[pl.ds(base, _BPW)], idx_v)

    def _bounce(off, size):
        pltpu.sync_copy(table_hbm.at[pl.ds(off, size)],
                        stage_v.at[pl.ds(0, size)])
        pltpu.sync_copy(stage_v.at[pl.ds(0, size)],
                        table_sp.at[pl.ds(off, size)])

    @pl.when(s < _NS - 1)
    def _stage_body():
        for j in range(_NFULL):
            _bounce(pl.multiple_of(s * _CHUNK + j * _SCHUNK, 8), _SCHUNK)
        _bounce(pl.multiple_of(s * _CHUNK + _NFULL * _SCHUNK, 8), _BTAIL)

    @pl.when(s == _NS - 1)
    def _stage_tail():
        for j in range(_NFULL):
            _bounce(15 * _CHUNK + j * _SCHUNK, _SCHUNK)
        _bounce(15 * _CHUNK + _NFULL * _SCHUNK, _TTAIL)

    plsc.subcore_barrier()

    pltpu.async_copy(table_sp.at[idx_v], vals_v, sem).wait()
    pltpu.sync_copy(vals_v, out_hbm.at[pl.ds(base, _BPW)])


def kernel(ind, scales):
    flat = ind.reshape(_B).astype(jnp.int32)
    out = _gather_sc(flat, scales)
    return out.reshape(_BATCH, _FIELDS)


# speedup vs baseline: 1.6011x; 1.1891x over previous
import functools

import jax
import jax.numpy as jnp
from jax import lax
from jax.experimental import pallas as pl
from jax.experimental.pallas import tpu as pltpu
from jax.experimental.pallas import tpu_sc as plsc

_BATCH = 16384
_FIELDS = 26
_V = 1000000
_NC = 2
_NS = 16
_NW = _NC * _NS
_ROWS = _BATCH // _NW
_BLK = 128
_NBLK = _ROWS // _BLK
_BE = _BLK * _FIELDS

_CHUNK = 62504
_TAIL = _V - 15 * _CHUNK
_SCHUNK = 13312
_NFULL = _CHUNK // _SCHUNK
_BTAIL = _CHUNK - _NFULL * _SCHUNK
_TTAIL = _TAIL - _NFULL * _SCHUNK

_mesh = plsc.VectorSubcoreMesh(core_axis_name="c", subcore_axis_name="s")


@functools.partial(
    pl.kernel,
    mesh=_mesh,
    out_type=jax.ShapeDtypeStruct((_BATCH, _FIELDS), jnp.float32),
    scratch_types=[
        pltpu.VMEM((_BLK, _FIELDS), jnp.int32),
        pltpu.VMEM((_BE,), jnp.int32),
        pltpu.VMEM((_BE,), jnp.float32),
        pltpu.VMEM((_BLK, _FIELDS), jnp.float32),
        pltpu.VMEM((_SCHUNK,), jnp.float32),
        pltpu.VMEM_SHARED((_V,), jnp.float32),
        pltpu.SemaphoreType.DMA,
    ],
)
def _gather_sc(idx_hbm, table_hbm, out_hbm, idx2_v, idx_v, vals_v, vals2_v,
               stage_v, table_sp, sem):
    s = lax.axis_index("s")
    wid = s * _NC + lax.axis_index("c")
    row0 = wid * _ROWS

    def _bounce(off, size):
        pltpu.sync_copy(table_hbm.at[pl.ds(off, size)],
                        stage_v.at[pl.ds(0, size)])
        pltpu.sync_copy(stage_v.at[pl.ds(0, size)],
                        table_sp.at[pl.ds(off, size)])

    @pl.when(s < _NS - 1)
    def _stage_body():
        for j in range(_NFULL):
            _bounce(pl.multiple_of(s * _CHUNK + j * _SCHUNK, 8), _SCHUNK)
        _bounce(pl.multiple_of(s * _CHUNK + _NFULL * _SCHUNK, 8), _BTAIL)

    @pl.when(s == _NS - 1)
    def _stage_tail():
        for j in range(_NFULL):
            _bounce(15 * _CHUNK + j * _SCHUNK, _SCHUNK)
        _bounce(15 * _CHUNK + _NFULL * _SCHUNK, _TTAIL)

    plsc.subcore_barrier()

    for blk in range(_NBLK):
        r0 = row0 + blk * _BLK
        pltpu.sync_copy(idx_hbm.at[pl.ds(r0, _BLK)], idx2_v)

        def _flat_row(r, _):
            a = idx2_v[r, pl.ds(0, 16)]
            b = idx2_v[r, pl.ds(10, 16)]
            idx_v[pl.ds(r * _FIELDS, 16)] = a
            idx_v[pl.ds(r * _FIELDS + 10, 16)] = b
            return _

        lax.fori_loop(0, _BLK, _flat_row, None)

        pltpu.async_copy(table_sp.at[idx_v], vals_v, sem).wait()

        def _unflat_row(r, _):
            a = vals_v[pl.ds(r * _FIELDS, 16)]
            b = vals_v[pl.ds(r * _FIELDS + 10, 16)]
            vals2_v[r, pl.ds(0, 16)] = a
            vals2_v[r, pl.ds(10, 16)] = b
            return _

        lax.fori_loop(0, _BLK, _unflat_row, None)
        pltpu.sync_copy(vals2_v, out_hbm.at[pl.ds(r0, _BLK)])


def kernel(ind, scales):
    return _gather_sc(ind.astype(jnp.int32), scales)


# speedup vs baseline: 1.7585x; 1.0983x over previous
import functools

import jax
import jax.numpy as jnp
from jax import lax
from jax.experimental import pallas as pl
from jax.experimental.pallas import tpu as pltpu
from jax.experimental.pallas import tpu_sc as plsc

_BATCH = 16384
_FIELDS = 26
_V = 1000000
_NC = 2
_NS = 16
_NW = _NC * _NS
_ROWS = _BATCH // _NW
_CROWS = 64
_NCH = _ROWS // _CROWS
_CE = _CROWS * _FIELDS

_CHUNK = 62504
_TAIL = _V - 15 * _CHUNK
_SCHUNK = 13312
_NFULL = _CHUNK // _SCHUNK
_BTAIL = _CHUNK - _NFULL * _SCHUNK
_TTAIL = _TAIL - _NFULL * _SCHUNK

_mesh = plsc.VectorSubcoreMesh(core_axis_name="c", subcore_axis_name="s")


@functools.partial(
    pl.kernel,
    mesh=_mesh,
    out_type=jax.ShapeDtypeStruct((_BATCH, _FIELDS), jnp.float32),
    scratch_types=[
        pltpu.VMEM((_CROWS, _FIELDS), jnp.int32),
        pltpu.VMEM((_ROWS * _FIELDS,), jnp.int32),
        pltpu.VMEM((_CE,), jnp.float32),
        pltpu.VMEM((_CE,), jnp.float32),
        pltpu.VMEM((_CROWS, _FIELDS), jnp.float32),
        pltpu.VMEM((_SCHUNK,), jnp.float32),
        pltpu.VMEM((_SCHUNK,), jnp.float32),
        pltpu.VMEM_SHARED((_V,), jnp.float32),
        pltpu.SemaphoreType.DMA,
        pltpu.SemaphoreType.DMA,
        pltpu.SemaphoreType.DMA,
        pltpu.SemaphoreType.DMA,
    ],
)
def _gather_sc(idx_hbm, table_hbm, out_hbm, idx2_v, idx_v, vals_a, vals_b,
               vals2_v, stage_a, stage_b, table_sp, sem1, sem2, gsem_a,
               gsem_b):
    s = lax.axis_index("s")
    wid = s * _NC + lax.axis_index("c")
    row0 = wid * _ROWS
    stage = (stage_a, stage_b)

    def _flatten_chunk(ch):
        pltpu.sync_copy(idx_hbm.at[pl.ds(row0 + ch * _CROWS, _CROWS)], idx2_v)

        def _rows8(g, _):
            r = g * 8
            for k in range(8):
                fo = (ch * _CROWS + r + k) * _FIELDS
                a = idx2_v[r + k, pl.ds(0, 16)]
                b = idx2_v[r + k, pl.ds(10, 16)]
                idx_v[pl.ds(fo, 16)] = a
                idx_v[pl.ds(fo + 10, 16)] = b
            return _

        lax.fori_loop(0, _CROWS // 8, _rows8, None)

    def _stage_pipeline(pieces):
        np_ = len(pieces)
        off0, sz0 = pieces[0]
        h1 = pltpu.async_copy(table_hbm.at[pl.ds(off0, sz0)],
                              stage[0].at[pl.ds(0, sz0)], sem1)
        ch = 0
        for j, (off, sz) in enumerate(pieces):
            h1.wait()
            h2 = pltpu.async_copy(stage[j % 2].at[pl.ds(0, sz)],
                                  table_sp.at[pl.ds(off, sz)], sem2)
            if j + 1 < np_:
                off_n, sz_n = pieces[j + 1]
                h1 = pltpu.async_copy(table_hbm.at[pl.ds(off_n, sz_n)],
                                      stage[(j + 1) % 2].at[pl.ds(0, sz_n)],
                                      sem1)
            if ch < _NCH:
                _flatten_chunk(ch)
                ch += 1
            h2.wait()
        while ch < _NCH:
            _flatten_chunk(ch)
            ch += 1

    @pl.when(s < _NS - 1)
    def _stage_body():
        base = pl.multiple_of(s * _CHUNK, 8)
        pieces = [(pl.multiple_of(base + j * _SCHUNK, 8), _SCHUNK)
                  for j in range(_NFULL)]
        pieces.append((pl.multiple_of(base + _NFULL * _SCHUNK, 8), _BTAIL))
        _stage_pipeline(pieces)

    @pl.when(s == _NS - 1)
    def _stage_tail():
        pieces = [(15 * _CHUNK + j * _SCHUNK, _SCHUNK)
                  for j in range(_NFULL)]
        pieces.append((15 * _CHUNK + _NFULL * _SCHUNK, _TTAIL))
        _stage_pipeline(pieces)

    plsc.subcore_barrier()

    vals = (vals_a, vals_b)
    gsem = (gsem_a, gsem_b)

    def _start_gather(ch):
        return pltpu.async_copy(
            table_sp.at[idx_v.at[pl.ds(ch * _CE, _CE)]], vals[ch % 2],
            gsem[ch % 2])

    def _unflat_store(ch):
        vb = vals[ch % 2]

        def _rows8(g, _):
            r = g * 8
            for k in range(8):
                fo = (r + k) * _FIELDS
                a = vb[pl.ds(fo, 16)]
                b = vb[pl.ds(fo + 10, 16)]
                vals2_v[r + k, pl.ds(0, 16)] = a
                vals2_v[r + k, pl.ds(10, 16)] = b
            return _

        lax.fori_loop(0, _CROWS // 8, _rows8, None)
        pltpu.sync_copy(vals2_v,
                        out_hbm.at[pl.ds(row0 + ch * _CROWS, _CROWS)])

    g = _start_gather(0)
    for ch in range(_NCH):
        g_next = _start_gather(ch + 1) if ch + 1 < _NCH else None
        g.wait()
        _unflat_store(ch)
        g = g_next


def kernel(ind, scales):
    return _gather_sc(ind.astype(jnp.int32), scales)
